# Initial kernel scaffold; baseline (speedup 1.0000x reference)
import jax
import jax.numpy as jnp
from jax.experimental import pallas as pl


def kernel(user, item, theta_table, a_table, b_table, c_table):
    raise NotImplementedError("write your pallas kernel here")


# speedup vs baseline: 1.1894x; 1.1894x over previous
import functools

import jax
import jax.numpy as jnp
from jax import lax
from jax.experimental import pallas as pl
from jax.experimental.pallas import tpu as pltpu
from jax.experimental.pallas import tpu_sc as plsc

BATCH = 16384
L = 16


def _sigmoid(x):
    return 1.0 / (1.0 + jnp.exp(-x))


def _body(nc, bpw, user_hbm, item_hbm, theta_hbm, a_hbm, b_hbm, c_hbm,
          out_hbm, uidx_v, iidx_v, th_v, a_v, b_v, c_v, out_v, sem):
    wid = lax.axis_index("s") * nc + lax.axis_index("c")
    base = wid * bpw
    pltpu.sync_copy(user_hbm.at[pl.ds(base, bpw)], uidx_v)
    pltpu.sync_copy(item_hbm.at[pl.ds(base, bpw)], iidx_v)
    cth = pltpu.async_copy(theta_hbm.at[uidx_v], th_v, sem)
    ca = pltpu.async_copy(a_hbm.at[iidx_v], a_v, sem)
    cb = pltpu.async_copy(b_hbm.at[iidx_v], b_v, sem)
    cc = pltpu.async_copy(c_hbm.at[iidx_v], c_v, sem)
    cth.wait()
    ca.wait()
    cb.wait()
    cc.wait()
    for i in range(bpw // L):
        sl = pl.ds(i * L, L)
        th = _sigmoid(th_v[sl]) - 0.5
        a = _sigmoid(a_v[sl])
        b = _sigmoid(b_v[sl]) - 0.5
        c = _sigmoid(c_v[sl])
        out_v[sl] = c + (1.0 - c) * _sigmoid(1.702 * a * (th - b))
    pltpu.sync_copy(out_v, out_hbm.at[pl.ds(base, bpw)])


def kernel(user, item, theta_table, a_table, b_table, c_table):
    info = plsc.get_sparse_core_info()
    nc, ns = info.num_cores, info.num_subcores
    nw = nc * ns
    bpw = BATCH // nw
    mesh = plsc.VectorSubcoreMesh(core_axis_name="c", subcore_axis_name="s")
    k = pl.kernel(
        functools.partial(_body, nc, bpw),
        mesh=mesh,
        out_type=jax.ShapeDtypeStruct((BATCH,), jnp.float32),
        scratch_types=[
            pltpu.VMEM((bpw,), jnp.int32),
            pltpu.VMEM((bpw,), jnp.int32),
            pltpu.VMEM((bpw,), jnp.float32),
            pltpu.VMEM((bpw,), jnp.float32),
            pltpu.VMEM((bpw,), jnp.float32),
            pltpu.VMEM((bpw,), jnp.float32),
            pltpu.VMEM((bpw,), jnp.float32),
            pltpu.SemaphoreType.DMA,
        ],
    )
    return k(user, item,
             jnp.reshape(theta_table, (-1,)),
             jnp.reshape(a_table, (-1,)),
             jnp.reshape(b_table, (-1,)),
             jnp.reshape(c_table, (-1,)))


# speedup vs baseline: 1.2003x; 1.0092x over previous
import functools

import jax
import jax.numpy as jnp
from jax import lax
from jax.experimental import pallas as pl
from jax.experimental.pallas import tpu as pltpu
from jax.experimental.pallas import tpu_sc as plsc

BATCH = 16384
L = 16


def _sigmoid(x):
    return 1.0 / (1.0 + jnp.exp(-x))


CHUNK = 128


def _body(nc, bpw, user_hbm, item_hbm, theta_hbm, a_hbm, b_hbm, c_hbm,
          out_hbm, uidx_v, iidx_v, th_v, a_v, b_v, c_v, out_v,
          isem, *gsems):
    wid = lax.axis_index("s") * nc + lax.axis_index("c")
    base = wid * bpw
    nchunk = bpw // CHUNK
    ciu = pltpu.async_copy(user_hbm.at[pl.ds(base, bpw)], uidx_v, isem)
    cii = pltpu.async_copy(item_hbm.at[pl.ds(base, bpw)], iidx_v, isem)
    ciu.wait()
    cii.wait()
    copies = []
    for j in range(nchunk):
        sl = pl.ds(j * CHUNK, CHUNK)
        copies.append((
            pltpu.async_copy(theta_hbm.at[uidx_v.at[sl]], th_v.at[sl], gsems[j]),
            pltpu.async_copy(a_hbm.at[iidx_v.at[sl]], a_v.at[sl], gsems[j]),
            pltpu.async_copy(b_hbm.at[iidx_v.at[sl]], b_v.at[sl], gsems[j]),
            pltpu.async_copy(c_hbm.at[iidx_v.at[sl]], c_v.at[sl], gsems[j]),
        ))
    for j in range(nchunk):
        for cp in copies[j]:
            cp.wait()
        for i in range(j * CHUNK // L, (j + 1) * CHUNK // L):
            sl = pl.ds(i * L, L)
            th = _sigmoid(th_v[sl]) - 0.5
            a = _sigmoid(a_v[sl])
            b = _sigmoid(b_v[sl]) - 0.5
            c = _sigmoid(c_v[sl])
            out_v[sl] = c + (1.0 - c) * _sigmoid(1.702 * a * (th - b))
    pltpu.sync_copy(out_v, out_hbm.at[pl.ds(base, bpw)])


def kernel(user, item, theta_table, a_table, b_table, c_table):
    info = plsc.get_sparse_core_info()
    nc, ns = info.num_cores, info.num_subcores
    nw = nc * ns
    bpw = BATCH // nw
    mesh = plsc.VectorSubcoreMesh(core_axis_name="c", subcore_axis_name="s")
    k = pl.kernel(
        functools.partial(_body, nc, bpw),
        mesh=mesh,
        out_type=jax.ShapeDtypeStruct((BATCH,), jnp.float32),
        scratch_types=[
            pltpu.VMEM((bpw,), jnp.int32),
            pltpu.VMEM((bpw,), jnp.int32),
            pltpu.VMEM((bpw,), jnp.float32),
            pltpu.VMEM((bpw,), jnp.float32),
            pltpu.VMEM((bpw,), jnp.float32),
            pltpu.VMEM((bpw,), jnp.float32),
            pltpu.VMEM((bpw,), jnp.float32),
            pltpu.SemaphoreType.DMA,
        ] + [pltpu.SemaphoreType.DMA] * (bpw // CHUNK),
    )
    return k(user, item,
             jnp.reshape(theta_table, (-1,)),
             jnp.reshape(a_table, (-1,)),
             jnp.reshape(b_table, (-1,)),
             jnp.reshape(c_table, (-1,)))


# speedup vs baseline: 1.2328x; 1.0271x over previous
import functools

import jax
import jax.numpy as jnp
from jax import lax
from jax.experimental import pallas as pl
from jax.experimental.pallas import tpu as pltpu
from jax.experimental.pallas import tpu_sc as plsc

BATCH = 16384
L = 16


def _sigmoid(x):
    return 1.0 / (1.0 + jnp.exp(-x))


def _body(nc, bpw, user_hbm, item_hbm, theta_hbm, a_hbm, b_hbm, c_hbm,
          out_hbm, uidx_v, iidx_v, th_v, a_v, b_v, c_v, out_v, isem, gsem):
    wid = lax.axis_index("s") * nc + lax.axis_index("c")
    base = wid * bpw
    ciu = pltpu.async_copy(user_hbm.at[pl.ds(base, bpw)], uidx_v, isem)
    cii = pltpu.async_copy(item_hbm.at[pl.ds(base, bpw)], iidx_v, isem)
    ciu.wait()
    cii.wait()
    cth = pltpu.async_copy(theta_hbm.at[uidx_v], th_v, gsem)
    ca = pltpu.async_copy(a_hbm.at[iidx_v], a_v, gsem)
    cb = pltpu.async_copy(b_hbm.at[iidx_v], b_v, gsem)
    cc = pltpu.async_copy(c_hbm.at[iidx_v], c_v, gsem)
    cth.wait()
    ca.wait()
    cb.wait()
    cc.wait()

    def step(i, carry):
        sl = pl.ds(i * L, L)
        th = _sigmoid(th_v[sl]) - 0.5
        a = _sigmoid(a_v[sl])
        b = _sigmoid(b_v[sl]) - 0.5
        c = _sigmoid(c_v[sl])
        out_v[sl] = c + (1.0 - c) * _sigmoid(1.702 * a * (th - b))
        return carry

    lax.fori_loop(0, bpw // L, step, 0)
    pltpu.sync_copy(out_v, out_hbm.at[pl.ds(base, bpw)])


def kernel(user, item, theta_table, a_table, b_table, c_table):
    info = plsc.get_sparse_core_info()
    nc, ns = info.num_cores, info.num_subcores
    bpw = BATCH // (nc * ns)
    mesh = plsc.VectorSubcoreMesh(core_axis_name="c", subcore_axis_name="s")
    k = pl.kernel(
        functools.partial(_body, nc, bpw),
        mesh=mesh,
        out_type=jax.ShapeDtypeStruct((BATCH,), jnp.float32),
        scratch_types=[
            pltpu.VMEM((bpw,), jnp.int32),
            pltpu.VMEM((bpw,), jnp.int32),
            pltpu.VMEM((bpw,), jnp.float32),
            pltpu.VMEM((bpw,), jnp.float32),
            pltpu.VMEM((bpw,), jnp.float32),
            pltpu.VMEM((bpw,), jnp.float32),
            pltpu.VMEM((bpw,), jnp.float32),
            pltpu.SemaphoreType.DMA,
            pltpu.SemaphoreType.DMA,
        ],
    )
    return k(user, item,
             jnp.reshape(theta_table, (-1,)),
             jnp.reshape(a_table, (-1,)),
             jnp.reshape(b_table, (-1,)),
             jnp.reshape(c_table, (-1,)))
